# baseline (device time: 42686 ns/iter reference)
import jax
import jax.numpy as jnp
from jax import lax
from jax.experimental import pallas as pl
from jax.experimental.pallas import tpu as pltpu

N_DEV = 4


def kernel(x, w_mat):
    k_glob, m_per = x.shape
    k_w, n = w_mat.shape
    blk = m_per

    def body(x_ref, w_ref, out_ref, comm_ref, send_sems, recv_sems):
        my = lax.axis_index("i")

        barrier = pltpu.get_barrier_semaphore()
        for o in (1, 2, 3):
            pl.semaphore_signal(
                barrier, inc=1,
                device_id=((my + o) % N_DEV,),
                device_id_type=pl.DeviceIdType.MESH,
            )
        pl.semaphore_wait(barrier, 3)

        rdmas = []
        for o in (1, 2, 3):
            dst = (my + o) % N_DEV
            s = N_DEV - o
            rdma = pltpu.make_async_remote_copy(
                src_ref=x_ref.at[pl.ds(dst * blk, blk), :],
                dst_ref=comm_ref.at[s - 1],
                send_sem=send_sems.at[o - 1],
                recv_sem=recv_sems.at[s - 1],
                device_id=(dst,),
                device_id_type=pl.DeviceIdType.MESH,
            )
            rdma.start()
            rdmas.append(rdma)

        acc = jnp.dot(
            x_ref[pl.ds(my * blk, blk), :],
            w_ref[pl.ds(my * blk, blk), :],
            preferred_element_type=jnp.float32,
        )

        for o, s in ((3, 1), (1, 3), (2, 2)):
            rdmas[o - 1].wait_recv()
            src_dev = (my + s) % N_DEV
            acc = acc + jnp.dot(
                comm_ref[s - 1],
                w_ref[pl.ds(src_dev * blk, blk), :],
                preferred_element_type=jnp.float32,
            )

        out_ref[:, :] = acc

        for r in rdmas:
            r.wait_send()

    return pl.pallas_call(
        body,
        out_shape=jax.ShapeDtypeStruct((m_per, n), jnp.float32),
        in_specs=[
            pl.BlockSpec(memory_space=pltpu.VMEM),
            pl.BlockSpec(memory_space=pltpu.VMEM),
        ],
        out_specs=pl.BlockSpec(memory_space=pltpu.VMEM),
        scratch_shapes=[
            pltpu.VMEM((N_DEV - 1, blk, m_per), jnp.float32),
            pltpu.SemaphoreType.DMA((N_DEV - 1,)),
            pltpu.SemaphoreType.DMA((N_DEV - 1,)),
        ],
        compiler_params=pltpu.CompilerParams(collective_id=0),
    )(x, w_mat)


# device time: 35806 ns/iter; 1.1921x vs baseline; 1.1921x over previous
import jax
import jax.numpy as jnp
from jax import lax
from jax.experimental import pallas as pl
from jax.experimental.pallas import tpu as pltpu

N_DEV = 4


def kernel(x, w_mat):
    k_glob, m_per = x.shape
    k_w, n = w_mat.shape
    blk = m_per

    def body(x_ref, w_hbm, out_ref, comm_ref, wbuf, send_sems, recv_sems,
             w_sems):
        my = lax.axis_index("i")

        barrier = pltpu.get_barrier_semaphore()
        for o in (1, 2, 3):
            pl.semaphore_signal(
                barrier, inc=1,
                device_id=((my + o) % N_DEV,),
                device_id_type=pl.DeviceIdType.MESH,
            )
        pl.semaphore_wait(barrier, 3)

        rdmas = []
        for o in (1, 2, 3):
            dst = (my + o) % N_DEV
            s = N_DEV - o
            rdma = pltpu.make_async_remote_copy(
                src_ref=x_ref.at[pl.ds(dst * blk, blk), :],
                dst_ref=comm_ref.at[s - 1],
                send_sem=send_sems.at[o - 1],
                recv_sem=recv_sems.at[s - 1],
                device_id=(dst,),
                device_id_type=pl.DeviceIdType.MESH,
            )
            rdma.start()
            rdmas.append(rdma)

        w_copies = []
        for c, off in enumerate((0, 1, 3, 2)):
            src_dev = (my + off) % N_DEV
            cp = pltpu.make_async_copy(
                w_hbm.at[pl.ds(src_dev * blk, blk), :],
                wbuf.at[c],
                w_sems.at[c],
            )
            cp.start()
            w_copies.append(cp)

        w_copies[0].wait()
        out_ref[:, :] = jnp.dot(
            x_ref[pl.ds(my * blk, blk), :], wbuf[0],
            preferred_element_type=jnp.float32,
        )

        for c, (o, s) in zip((1, 2, 3), ((3, 1), (1, 3), (2, 2))):
            rdmas[o - 1].wait_recv()
            w_copies[c].wait()
            out_ref[:, :] += jnp.dot(
                comm_ref[s - 1], wbuf[c],
                preferred_element_type=jnp.float32,
            )

        for r in rdmas:
            r.wait_send()

    return pl.pallas_call(
        body,
        out_shape=jax.ShapeDtypeStruct((m_per, n), jnp.float32),
        in_specs=[
            pl.BlockSpec(memory_space=pltpu.VMEM),
            pl.BlockSpec(memory_space=pltpu.MemorySpace.HBM),
        ],
        out_specs=pl.BlockSpec(memory_space=pltpu.VMEM),
        scratch_shapes=[
            pltpu.VMEM((N_DEV - 1, blk, m_per), jnp.float32),
            pltpu.VMEM((N_DEV, blk, n), jnp.float32),
            pltpu.SemaphoreType.DMA((N_DEV - 1,)),
            pltpu.SemaphoreType.DMA((N_DEV - 1,)),
            pltpu.SemaphoreType.DMA((N_DEV,)),
        ],
        compiler_params=pltpu.CompilerParams(collective_id=0),
    )(x, w_mat)


# device time: 24565 ns/iter; 1.7377x vs baseline; 1.4576x over previous
import jax
import jax.numpy as jnp
from jax import lax
from jax.experimental import pallas as pl
from jax.experimental.pallas import tpu as pltpu

N_DEV = 4


def kernel(x, w_mat):
    k_glob, m_per = x.shape
    k_w, n = w_mat.shape
    blk = m_per

    def body(x_ref, w_hbm, out_ref, xbf_ref, comm_ref, wbuf, send_sems,
             recv_sems, w_sems):
        my = lax.axis_index("i")

        barrier = pltpu.get_barrier_semaphore()
        for o in (1, 2, 3):
            pl.semaphore_signal(
                barrier, inc=1,
                device_id=((my + o) % N_DEV,),
                device_id_type=pl.DeviceIdType.MESH,
            )

        xbf_ref[:, :] = x_ref[:, :].astype(jnp.bfloat16)

        pl.semaphore_wait(barrier, 3)

        rdmas = []
        for o in (1, 2, 3):
            dst = (my + o) % N_DEV
            s = N_DEV - o
            rdma = pltpu.make_async_remote_copy(
                src_ref=xbf_ref.at[pl.ds(dst * blk, blk), :],
                dst_ref=comm_ref.at[s - 1],
                send_sem=send_sems.at[o - 1],
                recv_sem=recv_sems.at[s - 1],
                device_id=(dst,),
                device_id_type=pl.DeviceIdType.MESH,
            )
            rdma.start()
            rdmas.append(rdma)

        w_copies = []
        for c, off in enumerate((0, 1, 3, 2)):
            src_dev = (my + off) % N_DEV
            cp = pltpu.make_async_copy(
                w_hbm.at[pl.ds(src_dev * blk, blk), :],
                wbuf.at[c],
                w_sems.at[c],
            )
            cp.start()
            w_copies.append(cp)

        w_copies[0].wait()
        out_ref[:, :] = jnp.dot(
            x_ref[pl.ds(my * blk, blk), :], wbuf[0],
            preferred_element_type=jnp.float32,
        )

        for c, (o, s) in zip((1, 2, 3), ((3, 1), (1, 3), (2, 2))):
            rdmas[o - 1].wait_recv()
            w_copies[c].wait()
            out_ref[:, :] += jnp.dot(
                comm_ref[s - 1].astype(jnp.float32), wbuf[c],
                preferred_element_type=jnp.float32,
            )

        for r in rdmas:
            r.wait_send()

    return pl.pallas_call(
        body,
        out_shape=jax.ShapeDtypeStruct((m_per, n), jnp.float32),
        in_specs=[
            pl.BlockSpec(memory_space=pltpu.VMEM),
            pl.BlockSpec(memory_space=pltpu.MemorySpace.HBM),
        ],
        out_specs=pl.BlockSpec(memory_space=pltpu.VMEM),
        scratch_shapes=[
            pltpu.VMEM((k_glob, m_per), jnp.bfloat16),
            pltpu.VMEM((N_DEV - 1, blk, m_per), jnp.bfloat16),
            pltpu.VMEM((N_DEV, blk, n), jnp.float32),
            pltpu.SemaphoreType.DMA((N_DEV - 1,)),
            pltpu.SemaphoreType.DMA((N_DEV - 1,)),
            pltpu.SemaphoreType.DMA((N_DEV,)),
        ],
        compiler_params=pltpu.CompilerParams(collective_id=0),
    )(x, w_mat)


# device time: 22311 ns/iter; 1.9132x vs baseline; 1.1010x over previous
import jax
import jax.numpy as jnp
from jax import lax
from jax.experimental import pallas as pl
from jax.experimental.pallas import tpu as pltpu

N_DEV = 4


def kernel(x, w_mat):
    k_glob, m_per = x.shape
    k_w, n = w_mat.shape
    blk = m_per

    def body(x_ref, w_hbm, out_ref, xbf_ref, comm_ref, wbuf, send_sems,
             recv_sems, w_sems):
        my = lax.axis_index("i")

        barrier = pltpu.get_barrier_semaphore()
        for o in (1, 2, 3):
            pl.semaphore_signal(
                barrier, inc=1,
                device_id=((my + o) % N_DEV,),
                device_id_type=pl.DeviceIdType.MESH,
            )

        w_copies = []
        for c, off in enumerate((0, 1, 3, 2)):
            src_dev = (my + off) % N_DEV
            cp = pltpu.make_async_copy(
                w_hbm.at[pl.ds(src_dev * blk, blk), :],
                wbuf.at[c],
                w_sems.at[c],
            )
            cp.start()
            w_copies.append(cp)

        rdmas = []
        for o in (1, 2, 3):
            dst = (my + o) % N_DEV
            s = N_DEV - o
            rdmas.append(pltpu.make_async_remote_copy(
                src_ref=xbf_ref.at[o - 1],
                dst_ref=comm_ref.at[s - 1],
                send_sem=send_sems.at[o - 1],
                recv_sem=recv_sems.at[s - 1],
                device_id=(dst,),
                device_id_type=pl.DeviceIdType.MESH,
            ))

        for o in (1, 3):
            dst = (my + o) % N_DEV
            xbf_ref[o - 1, :, :] = (
                x_ref[pl.ds(dst * blk, blk), :].astype(jnp.bfloat16))
            if o == 1:
                pl.semaphore_wait(barrier, 3)
            rdmas[o - 1].start()
        diag = (my + 2) % N_DEV
        xbf_ref[1, :, :] = x_ref[pl.ds(diag * blk, blk), :].astype(jnp.bfloat16)

        w_copies[0].wait()
        out_ref[:, :] = jnp.dot(
            x_ref[pl.ds(my * blk, blk), :], wbuf[0],
            preferred_element_type=jnp.float32,
        )

        rdmas[0].wait_send()
        rdmas[2].wait_send()
        rdmas[1].start()

        for c, (o, s) in zip((1, 2, 3), ((3, 1), (1, 3), (2, 2))):
            rdmas[o - 1].wait_recv()
            w_copies[c].wait()
            out_ref[:, :] += jnp.dot(
                comm_ref[s - 1].astype(jnp.float32), wbuf[c],
                preferred_element_type=jnp.float32,
            )

        rdmas[1].wait_send()

    return pl.pallas_call(
        body,
        out_shape=jax.ShapeDtypeStruct((m_per, n), jnp.float32),
        in_specs=[
            pl.BlockSpec(memory_space=pltpu.VMEM),
            pl.BlockSpec(memory_space=pltpu.MemorySpace.HBM),
        ],
        out_specs=pl.BlockSpec(memory_space=pltpu.VMEM),
        scratch_shapes=[
            pltpu.VMEM((N_DEV - 1, blk, m_per), jnp.bfloat16),
            pltpu.VMEM((N_DEV - 1, blk, m_per), jnp.bfloat16),
            pltpu.VMEM((N_DEV, blk, n), jnp.float32),
            pltpu.SemaphoreType.DMA((N_DEV - 1,)),
            pltpu.SemaphoreType.DMA((N_DEV - 1,)),
            pltpu.SemaphoreType.DMA((N_DEV,)),
        ],
        compiler_params=pltpu.CompilerParams(collective_id=0),
    )(x, w_mat)


# device time: 21794 ns/iter; 1.9586x vs baseline; 1.0237x over previous
import jax
import jax.numpy as jnp
from jax import lax
from jax.experimental import pallas as pl
from jax.experimental.pallas import tpu as pltpu

N_DEV = 4


def kernel(x, w_mat):
    k_glob, m_per = x.shape
    k_w, n = w_mat.shape
    blk = m_per
    half = blk // 2

    def body(x_hbm, w_hbm, out_ref, xblk_ref, xbf_ref, comm_ref, wbuf,
             x_sems, w_sems, send_sems, recv_sems):
        my = lax.axis_index("i")

        barrier = pltpu.get_barrier_semaphore()
        for o in (1, 2, 3):
            pl.semaphore_signal(
                barrier, inc=1,
                device_id=((my + o) % N_DEV,),
                device_id_type=pl.DeviceIdType.MESH,
            )

        x_copies = {}
        for c, off in enumerate((1, 3, 2, 0)):
            src_dev = (my + off) % N_DEV
            cp = pltpu.make_async_copy(
                x_hbm.at[pl.ds(src_dev * blk, blk), :],
                xblk_ref.at[c],
                x_sems.at[c],
            )
            cp.start()
            x_copies[off] = (cp, c)

        w_copies = []
        for c, off in enumerate((0, 1, 3, 2)):
            src_dev = (my + off) % N_DEV
            cp = pltpu.make_async_copy(
                w_hbm.at[pl.ds(src_dev * blk, blk), :],
                wbuf.at[c],
                w_sems.at[c],
            )
            cp.start()
            w_copies.append(cp)

        def rdma_to(o, src, dst_slice, send_idx, recv_idx):
            return pltpu.make_async_remote_copy(
                src_ref=src,
                dst_ref=dst_slice,
                send_sem=send_sems.at[send_idx],
                recv_sem=recv_sems.at[recv_idx],
                device_id=((my + o) % N_DEV,),
                device_id_type=pl.DeviceIdType.MESH,
            )

        rdma_r = rdma_to(1, xbf_ref.at[0], comm_ref.at[2], 0, 2)
        rdma_l = rdma_to(3, xbf_ref.at[2], comm_ref.at[0], 2, 0)
        rdma_d1 = rdma_to(2, xbf_ref.at[1, pl.ds(0, half), :],
                          comm_ref.at[1, pl.ds(0, half), :], 1, 1)
        rdma_d2 = rdma_to(2, xbf_ref.at[1, pl.ds(half, half), :],
                          comm_ref.at[1, pl.ds(half, half), :], 3, 3)

        x_copies[1][0].wait()
        xbf_ref[0, :, :] = xblk_ref[x_copies[1][1]].astype(jnp.bfloat16)
        pl.semaphore_wait(barrier, 3)
        rdma_r.start()
        x_copies[3][0].wait()
        xbf_ref[2, :, :] = xblk_ref[x_copies[3][1]].astype(jnp.bfloat16)
        rdma_l.start()
        x_copies[2][0].wait()
        xbf_ref[1, :, :] = xblk_ref[x_copies[2][1]].astype(jnp.bfloat16)

        x_copies[0][0].wait()
        w_copies[0].wait()
        out_ref[:, :] = jnp.dot(
            xblk_ref[x_copies[0][1]], wbuf[0],
            preferred_element_type=jnp.float32,
        )

        rdma_r.wait_send()
        rdma_l.wait_send()
        rdma_d1.start()
        rdma_d2.start()

        for c, slot, rd in ((1, 0, rdma_l), (2, 2, rdma_r)):
            rd.wait_recv()
            w_copies[c].wait()
            out_ref[:, :] += jnp.dot(
                comm_ref[slot].astype(jnp.float32), wbuf[c],
                preferred_element_type=jnp.float32,
            )

        w_copies[3].wait()
        rdma_d1.wait_recv()
        out_ref[pl.ds(0, half), :] += jnp.dot(
            comm_ref[1, pl.ds(0, half), :].astype(jnp.float32),
            wbuf[3],
            preferred_element_type=jnp.float32,
        )
        rdma_d2.wait_recv()
        out_ref[pl.ds(half, half), :] += jnp.dot(
            comm_ref[1, pl.ds(half, half), :].astype(jnp.float32),
            wbuf[3],
            preferred_element_type=jnp.float32,
        )

        rdma_d1.wait_send()
        rdma_d2.wait_send()

    return pl.pallas_call(
        body,
        out_shape=jax.ShapeDtypeStruct((m_per, n), jnp.float32),
        in_specs=[
            pl.BlockSpec(memory_space=pltpu.MemorySpace.HBM),
            pl.BlockSpec(memory_space=pltpu.MemorySpace.HBM),
        ],
        out_specs=pl.BlockSpec(memory_space=pltpu.VMEM),
        scratch_shapes=[
            pltpu.VMEM((N_DEV, blk, m_per), jnp.float32),
            pltpu.VMEM((N_DEV - 1, blk, m_per), jnp.bfloat16),
            pltpu.VMEM((N_DEV - 1, blk, m_per), jnp.bfloat16),
            pltpu.VMEM((N_DEV, blk, n), jnp.float32),
            pltpu.SemaphoreType.DMA((N_DEV,)),
            pltpu.SemaphoreType.DMA((N_DEV,)),
            pltpu.SemaphoreType.DMA((N_DEV,)),
            pltpu.SemaphoreType.DMA((N_DEV,)),
        ],
        compiler_params=pltpu.CompilerParams(collective_id=0),
    )(x, w_mat)
